# baseline (speedup 1.0000x reference)
import functools

import jax
import jax.numpy as jnp
from jax import lax
from jax.experimental import pallas as pl
from jax.experimental.pallas import tpu as pltpu
from jax.experimental.pallas import tpu_sc as plsc

_NUM_FIELDS = 26
_FIELD_DIM = 100000
_EMBED_DIM = 16
_BATCH = 4096
_TABLE_ROWS = _NUM_FIELDS * _FIELD_DIM

_NC, _NS, _L = 2, 16, 16
_NW = _NC * _NS
_PAIRS = _NUM_FIELDS * _EMBED_DIM
_PPW = _PAIRS // _NW
_CHUNK = 128
_NCHUNK = _BATCH // _CHUNK

_mesh = plsc.VectorSubcoreMesh(
    core_axis_name="c", subcore_axis_name="s", num_cores=_NC, num_subcores=_NS
)


@functools.partial(
    pl.kernel,
    out_type=jax.ShapeDtypeStruct((_NUM_FIELDS * _EMBED_DIM * _BATCH,), jnp.float32),
    mesh=_mesh,
    scratch_types=[
        pltpu.VMEM((_BATCH,), jnp.int32),
        pltpu.VMEM((_BATCH,), jnp.float32),
        pltpu.SemaphoreType.DMA,
        pltpu.SemaphoreType.DMA,
    ],
    compiler_params=pltpu.CompilerParams(use_tc_tiling_on_sc=False),
)
def _embed_gather(xt_hbm, table_hbm, out_hbm, idx_v, row_v, gsem, osem):
    wid = lax.axis_index("s") * _NC + lax.axis_index("c")

    def body(k, _):
        p = wid * _PPW + k
        f = p // _EMBED_DIM
        d = lax.rem(p, _EMBED_DIM)
        pltpu.sync_copy(xt_hbm.at[pl.ds(f * _BATCH, _BATCH)], idx_v)
        base = d * _TABLE_ROWS + f * _FIELD_DIM
        base = pl.multiple_of(base, 8)
        copies = [
            pltpu.async_copy(
                table_hbm.at[pl.ds(base, _FIELD_DIM)].at[
                    idx_v.at[pl.ds(j * _CHUNK, _CHUNK)]
                ],
                row_v.at[pl.ds(j * _CHUNK, _CHUNK)],
                gsem,
            )
            for j in range(_NCHUNK)
        ]
        for cp in copies:
            cp.wait()
        pltpu.sync_copy(row_v, out_hbm.at[pl.ds(p * _BATCH, _BATCH)])
        return 0

    lax.fori_loop(0, _PPW, body, 0)


def kernel(x, table):
    xt_flat = jnp.swapaxes(x, 0, 1).reshape(_NUM_FIELDS * _BATCH)
    tt_flat = jnp.swapaxes(table, 0, 1).reshape(_EMBED_DIM * _TABLE_ROWS)
    out = _embed_gather(xt_flat, tt_flat)
    out = out.reshape(_NUM_FIELDS, _EMBED_DIM, _BATCH)
    return jnp.transpose(out, (2, 0, 1))


# speedup vs baseline: 1.0004x; 1.0004x over previous
import functools

import jax
import jax.numpy as jnp
from jax import lax
from jax.experimental import pallas as pl
from jax.experimental.pallas import tpu as pltpu
from jax.experimental.pallas import tpu_sc as plsc

_NUM_FIELDS = 26
_FIELD_DIM = 100000
_EMBED_DIM = 16
_BATCH = 4096
_TABLE_ROWS = _NUM_FIELDS * _FIELD_DIM

_NC, _NS, _L = 2, 16, 16
_NW = _NC * _NS
_PAIRS = _NUM_FIELDS * _EMBED_DIM
_PPW = _PAIRS // _NW
_UNROLL = 8

_mesh = plsc.VectorSubcoreMesh(
    core_axis_name="c", subcore_axis_name="s", num_cores=_NC, num_subcores=_NS
)


@functools.partial(
    pl.kernel,
    out_type=jax.ShapeDtypeStruct((_NUM_FIELDS * _EMBED_DIM * _BATCH,), jnp.float32),
    mesh=_mesh,
    scratch_types=[
        pltpu.VMEM((_BATCH,), jnp.int32),
        pltpu.VMEM((_BATCH,), jnp.float32),
        pltpu.SemaphoreType.DMA,
    ],
    compiler_params=pltpu.CompilerParams(use_tc_tiling_on_sc=False),
)
def _embed_gather(xt_hbm, table_hbm, out_hbm, idx_v, row_v, gsem):
    wid = lax.axis_index("s") * _NC + lax.axis_index("c")

    def pair_body(k, _):
        p = wid * _PPW + k
        f = p // _EMBED_DIM
        d = lax.rem(p, _EMBED_DIM)
        pltpu.sync_copy(xt_hbm.at[pl.ds(f * _BATCH, _BATCH)], idx_v)
        base = d * _TABLE_ROWS + f * _FIELD_DIM

        def gather_body(j, _):
            for u in range(_UNROLL):
                s = (j * _UNROLL + u) * _L
                idx16 = idx_v[pl.ds(s, _L)] + base
                pltpu.make_async_copy(
                    table_hbm.at[idx16], row_v.at[pl.ds(s, _L)], gsem
                ).start()
            return 0

        lax.fori_loop(0, _BATCH // _L // _UNROLL, gather_body, 0)
        pltpu.make_async_copy(
            table_hbm.at[pl.ds(0, _BATCH)], row_v, gsem
        ).wait()
        pltpu.sync_copy(row_v, out_hbm.at[pl.ds(p * _BATCH, _BATCH)])
        return 0

    lax.fori_loop(0, _PPW, pair_body, 0)


def kernel(x, table):
    xt_flat = jnp.swapaxes(x, 0, 1).reshape(_NUM_FIELDS * _BATCH)
    tt_flat = jnp.swapaxes(table, 0, 1).reshape(_EMBED_DIM * _TABLE_ROWS)
    out = _embed_gather(xt_flat, tt_flat)
    out = out.reshape(_NUM_FIELDS, _EMBED_DIM, _BATCH)
    return jnp.transpose(out, (2, 0, 1))


# speedup vs baseline: 1.0045x; 1.0041x over previous
import functools

import jax
import jax.numpy as jnp
from jax import lax
from jax.experimental import pallas as pl
from jax.experimental.pallas import tpu as pltpu
from jax.experimental.pallas import tpu_sc as plsc

_NUM_FIELDS = 26
_FIELD_DIM = 100000
_EMBED_DIM = 16
_BATCH = 4096
_TABLE_ROWS = _NUM_FIELDS * _FIELD_DIM

_NC, _NS, _L = 2, 16, 16
_NW = _NC * _NS
_PAIRS = _NUM_FIELDS * _EMBED_DIM
_PPW = _PAIRS // _NW
_CHUNK = 128
_NCHUNK = _BATCH // _CHUNK

_mesh = plsc.VectorSubcoreMesh(
    core_axis_name="c", subcore_axis_name="s", num_cores=_NC, num_subcores=_NS
)


@functools.partial(
    pl.kernel,
    out_type=jax.ShapeDtypeStruct((_NUM_FIELDS * _EMBED_DIM * _BATCH,), jnp.float32),
    mesh=_mesh,
    scratch_types=[
        pltpu.VMEM((2, _BATCH), jnp.int32),
        pltpu.VMEM((2, _BATCH), jnp.float32),
        pltpu.SemaphoreType.DMA,
        pltpu.SemaphoreType.DMA,
        pltpu.SemaphoreType.DMA,
    ],
    compiler_params=pltpu.CompilerParams(use_tc_tiling_on_sc=False),
)
def _embed_gather(xt_hbm, table_hbm, out_hbm, idx_v, row_v, gsem0, gsem1, osem):
    gsems = (gsem0, gsem1)
    wid = lax.axis_index("s") * _NC + lax.axis_index("c")
    p0 = wid * _PPW

    def fire(k):
        b = k % 2
        p = p0 + k
        f = p // _EMBED_DIM
        d = lax.rem(p, _EMBED_DIM)
        pltpu.sync_copy(xt_hbm.at[pl.ds(f * _BATCH, _BATCH)], idx_v.at[b])
        base = d * _TABLE_ROWS + f * _FIELD_DIM
        base = pl.multiple_of(base, 8)
        for j in range(_NCHUNK):
            pltpu.make_async_copy(
                table_hbm.at[pl.ds(base, _FIELD_DIM)].at[
                    idx_v.at[b, pl.ds(j * _CHUNK, _CHUNK)]
                ],
                row_v.at[b, pl.ds(j * _CHUNK, _CHUNK)],
                gsems[b],
            ).start()
        return p

    def drain_store(k, p):
        b = k % 2
        pltpu.make_async_copy(
            table_hbm.at[pl.ds(0, _BATCH)], row_v.at[b], gsems[b]
        ).wait()
        pltpu.async_copy(
            row_v.at[b], out_hbm.at[pl.ds(p * _BATCH, _BATCH)], osem
        ).wait()

    prev_p = fire(0)
    for k in range(1, _PPW):
        p = fire(k)
        drain_store(k - 1, prev_p)
        prev_p = p
    drain_store(_PPW - 1, prev_p)


def kernel(x, table):
    xt_flat = jnp.swapaxes(x, 0, 1).reshape(_NUM_FIELDS * _BATCH)
    tt_flat = jnp.swapaxes(table, 0, 1).reshape(_EMBED_DIM * _TABLE_ROWS)
    out = _embed_gather(xt_flat, tt_flat)
    out = out.reshape(_NUM_FIELDS, _EMBED_DIM, _BATCH)
    return jnp.transpose(out, (2, 0, 1))
